# baseline (speedup 1.0000x reference)
import functools

import jax
import jax.numpy as jnp
from jax import lax
from jax.experimental import pallas as pl
from jax.experimental.pallas import tpu as pltpu

B = 16
N = 8192
M = 1024
BBLK = 8


def _fps_kernel(x_ref, y_ref, z_ref, cx_ref, cy_ref, cz_ref, d_ref):
    d_ref[...] = jnp.full((BBLK, N), jnp.inf, dtype=jnp.float32)
    iota = lax.broadcasted_iota(jnp.int32, (BBLK, N), 1)

    def body(k, carry):
        fx, fy, fz = carry
        cx_ref[0, pl.ds(k, 1), :] = fx.reshape(1, BBLK)
        cy_ref[0, pl.ds(k, 1), :] = fy.reshape(1, BBLK)
        cz_ref[0, pl.ds(k, 1), :] = fz.reshape(1, BBLK)

        dx = x_ref[...] - fx
        dy = y_ref[...] - fy
        dz = z_ref[...] - fz
        dist = (dx * dx + dz * dz) + dy * dy
        d = jnp.minimum(d_ref[...], dist)
        d_ref[...] = d

        m = jnp.max(d, axis=1, keepdims=True)
        cand = jnp.where(d == m, iota, N)
        j = jnp.min(cand, axis=1, keepdims=True)
        onehot = iota == j
        zero = jnp.zeros((BBLK, N), dtype=jnp.float32)
        nfx = jnp.sum(jnp.where(onehot, x_ref[...], zero), axis=1, keepdims=True)
        nfy = jnp.sum(jnp.where(onehot, y_ref[...], zero), axis=1, keepdims=True)
        nfz = jnp.sum(jnp.where(onehot, z_ref[...], zero), axis=1, keepdims=True)
        return nfx, nfy, nfz

    init = (x_ref[:, 0:1], y_ref[:, 0:1], z_ref[:, 0:1])
    lax.fori_loop(0, M, body, init)


@jax.jit
def kernel(xyz):
    x = xyz[:, :, 0]
    y = xyz[:, :, 1]
    z = xyz[:, :, 2]
    ncell = B // BBLK
    out_shape = jax.ShapeDtypeStruct((ncell, M, BBLK), jnp.float32)
    cx, cy, cz = pl.pallas_call(
        _fps_kernel,
        grid=(ncell,),
        in_specs=[pl.BlockSpec((BBLK, N), lambda i: (i, 0))] * 3,
        out_specs=(pl.BlockSpec((1, M, BBLK), lambda i: (i, 0, 0)),) * 3,
        out_shape=(out_shape, out_shape, out_shape),
        scratch_shapes=[pltpu.VMEM((BBLK, N), jnp.float32)],
        compiler_params=pltpu.CompilerParams(
            dimension_semantics=("parallel",)
        ),
    )(x, y, z)
    cx = cx.transpose(0, 2, 1).reshape(B, M)
    cy = cy.transpose(0, 2, 1).reshape(B, M)
    cz = cz.transpose(0, 2, 1).reshape(B, M)
    return jnp.stack([cx, cy, cz], axis=-1)


# speedup vs baseline: 1.2940x; 1.2940x over previous
import jax
import jax.numpy as jnp
from jax import lax
from jax.experimental import pallas as pl
from jax.experimental.pallas import tpu as pltpu

B = 16
N = 8192
M = 1024


def _fps_kernel(x_ref, y_ref, z_ref, cx_ref, cy_ref, cz_ref):
    x = x_ref[...]
    y = y_ref[...]
    z = z_ref[...]
    iota = lax.broadcasted_iota(jnp.int32, (B, N), 1)

    def body(k, carry):
        d_prev, fx, fy, fz = carry
        cx_ref[pl.ds(k, 1), :] = fx.reshape(1, B)
        cy_ref[pl.ds(k, 1), :] = fy.reshape(1, B)
        cz_ref[pl.ds(k, 1), :] = fz.reshape(1, B)

        dx = x - fx
        dy = y - fy
        dz = z - fz
        dist = (dx * dx + dz * dz) + dy * dy
        d = jnp.minimum(d_prev, dist)

        dw, iw, xw, yw, zw = d, iota, x, y, z
        w = N
        while w > 1:
            h = w // 2
            da, db = dw[:, :h], dw[:, h:w]
            ia, ib = iw[:, :h], iw[:, h:w]
            keep_a = (da > db) | ((da == db) & (ia < ib))
            dw = jnp.where(keep_a, da, db)
            iw = jnp.where(keep_a, ia, ib)
            xw = jnp.where(keep_a, xw[:, :h], xw[:, h:w])
            yw = jnp.where(keep_a, yw[:, :h], yw[:, h:w])
            zw = jnp.where(keep_a, zw[:, :h], zw[:, h:w])
            w = h
        return d, xw, yw, zw

    init = (
        jnp.full((B, N), jnp.inf, dtype=jnp.float32),
        x[:, 0:1],
        y[:, 0:1],
        z[:, 0:1],
    )
    lax.fori_loop(0, M, body, init)


@jax.jit
def kernel(xyz):
    x = xyz[:, :, 0]
    y = xyz[:, :, 1]
    z = xyz[:, :, 2]
    out_shape = jax.ShapeDtypeStruct((M, B), jnp.float32)
    cx, cy, cz = pl.pallas_call(
        _fps_kernel,
        out_shape=(out_shape, out_shape, out_shape),
    )(x, y, z)
    return jnp.stack([cx.T, cy.T, cz.T], axis=-1)
